# baseline (speedup 1.0000x reference)
import functools

import jax
import jax.numpy as jnp
from jax import lax
from jax.experimental import pallas as pl
from jax.experimental.pallas import tpu as pltpu
from jax.experimental.pallas import tpu_sc as plsc

N = 10000
E = 160000
G = 64
BERT = 768
N2V = 128
FUS = 256
HID = 256
NC = 10

NP = 10240
EP = 163840
CNTP = 12288
F2 = 128
NT = NP // 16
CHUNK = 64
CH = EP // 16 // CHUNK

_mesh = plsc.VectorSubcoreMesh(core_axis_name="c", subcore_axis_name="s")


@functools.partial(
    pl.kernel,
    out_type=(
        jax.ShapeDtypeStruct((2, NP, 128), jnp.float32),
        jax.ShapeDtypeStruct((2, 128, 128), jnp.float32),
    ),
    mesh=_mesh,
    scratch_types=[
        pltpu.VMEM((128,), jnp.int32),
        pltpu.VMEM((128, 128), jnp.float32),
        pltpu.VMEM((128, 128), jnp.float32),
        pltpu.VMEM_SHARED((NP, 128), jnp.float32),
        pltpu.VMEM_SHARED((128, 128), jnp.float32),
    ],
)
def _deg_kernel(dst1d, bc1d, ones_h, zeros_h, degp, cntp,
                ibuf, onesv, zerosv, dacc, cacc):
    c = lax.axis_index("c")
    s = lax.axis_index("s")
    w = c * 16 + s
    pltpu.sync_copy(ones_h, onesv)
    pltpu.sync_copy(zeros_h, zerosv)
    for k in range(NT // 128):
        pltpu.sync_copy(zerosv, dacc.at[pl.ds(s * NT + k * 128, 128)])

    @pl.when(s == 0)
    def _():
        pltpu.sync_copy(zerosv, cacc)

    plsc.subcore_barrier()

    def dbody(j, carry):
        pltpu.sync_copy(dst1d.at[pl.ds(w * (EP // 32) + j * 128, 128)], ibuf)
        pltpu.sync_copy(onesv, dacc.at[ibuf], add=True)
        return carry

    lax.fori_loop(0, EP // 32 // 128, dbody, 0)

    def cbody(j, carry):
        pltpu.sync_copy(bc1d.at[pl.ds(w * (CNTP // 32) + j * 128, 128)], ibuf)
        pltpu.sync_copy(onesv, cacc.at[ibuf], add=True)
        return carry

    lax.fori_loop(0, CNTP // 32 // 128, cbody, 0)
    plsc.subcore_barrier()
    pltpu.sync_copy(dacc.at[pl.ds(s * NT, NT)], degp.at[c, pl.ds(s * NT, NT)])

    @pl.when(s == 0)
    def _():
        pltpu.sync_copy(cacc, cntp.at[c])


NBUF = 4

@functools.partial(
    pl.kernel,
    out_type=jax.ShapeDtypeStruct((2, NP, F2), jnp.float32),
    mesh=_mesh,
    scratch_types=[
        [pltpu.VMEM((CHUNK,), jnp.int32) for _ in range(NBUF)],
        [pltpu.VMEM((CHUNK,), jnp.int32) for _ in range(NBUF)],
        [pltpu.VMEM((CHUNK, F2), jnp.float32) for _ in range(NBUF)],
        [pltpu.SemaphoreType.DMA for _ in range(NBUF)],
        pltpu.VMEM_SHARED((NP, F2), jnp.float32),
    ],
)
def _scatter_kernel(tflat, srcoff, dst1d, s_out, sibuf, dibuf, rows, gsem, acc):
    c = lax.axis_index("c")
    s = lax.axis_index("s")
    ebase = s * (CH * CHUNK)

    def load_idx(b, j):
        pltpu.sync_copy(srcoff.at[pl.ds(c * EP + ebase + j * CHUNK, CHUNK)], sibuf[b])
        pltpu.sync_copy(dst1d.at[pl.ds(ebase + j * CHUNK, CHUNK)], dibuf[b])

    for b in range(NBUF):
        load_idx(b, b)
        pltpu.async_copy(tflat.at[sibuf[b]], rows[b], gsem[b])
    pltpu.sync_copy(tflat.at[pl.ds(c * NP + s * NT, NT)], acc.at[pl.ds(s * NT, NT)])
    plsc.subcore_barrier()

    def body(jj, carry):
        for b in range(NBUF):
            j = jj * NBUF + b
            pltpu.make_async_copy(tflat.at[sibuf[b]], rows[b], gsem[b]).wait()
            pltpu.sync_copy(rows[b], acc.at[dibuf[b]], add=True)
            load_idx(b, j + NBUF)
            pltpu.async_copy(tflat.at[sibuf[b]], rows[b], gsem[b])
        return carry

    lax.fori_loop(0, CH // NBUF - 1, body, 0)
    for b in range(NBUF):
        pltpu.make_async_copy(tflat.at[sibuf[b]], rows[b], gsem[b]).wait()
        pltpu.sync_copy(rows[b], acc.at[dibuf[b]], add=True)
    plsc.subcore_barrier()
    pltpu.sync_copy(acc.at[pl.ds(s * NT, NT)], s_out.at[c, pl.ds(s * NT, NT)])


@functools.partial(
    pl.kernel,
    out_type=jax.ShapeDtypeStruct((2, G, F2), jnp.float32),
    mesh=_mesh,
    scratch_types=[
        pltpu.VMEM((128,), jnp.int32),
        pltpu.VMEM((128, F2), jnp.float32),
        pltpu.VMEM_SHARED((128, F2), jnp.float32),
    ],
)
def _pool_kernel(hflat, bp1d, zeros_h, sums, ibuf, rows, pacc):
    c = lax.axis_index("c")
    s = lax.axis_index("s")

    @pl.when(s == 0)
    def _():
        pltpu.sync_copy(zeros_h, pacc)

    plsc.subcore_barrier()

    def body(j, carry):
        base = s * NT + j * 128
        pltpu.sync_copy(hflat.at[pl.ds(c * NP + base, 128)], rows)
        pltpu.sync_copy(bp1d.at[pl.ds(base, 128)], ibuf)
        pltpu.sync_copy(rows, pacc.at[ibuf], add=True)
        return carry

    lax.fori_loop(0, NT // 128, body, 0)
    plsc.subcore_barrier()

    @pl.when(s == 0)
    def _():
        pltpu.sync_copy(pacc.at[pl.ds(0, G)], sums.at[c])


def _fusion_body(x_ref, se_ref, w1_ref, b1_ref, w2_ref, b2_ref, wa_ref, ba_ref, out_ref):
    h1 = jnp.dot(x_ref[...], w1_ref[...], preferred_element_type=jnp.float32) + b1_ref[...]
    h2 = jnp.dot(se_ref[...], w2_ref[...], preferred_element_type=jnp.float32) + b2_ref[...]
    a1 = jnp.sum(h1 * wa_ref[...], axis=1, keepdims=True) + ba_ref[...]
    a2 = jnp.sum(h2 * wa_ref[...], axis=1, keepdims=True) + ba_ref[...]
    m = jnp.maximum(a1, a2)
    e1 = jnp.exp(a1 - m)
    e2 = jnp.exp(a2 - m)
    out_ref[...] = (e1 * h1 + e2 * h2) / (e1 + e2)


def _dinv_body(dp_ref, out_ref):
    deg = jnp.sum(dp_ref[...], axis=(0, 2)) * (1.0 / 128.0) + 1.0
    out_ref[...] = jnp.broadcast_to(lax.rsqrt(deg)[:, None], out_ref.shape)


def _t1_body(f_ref, di_ref, wg_ref, out_ref):
    dinv = di_ref[...][:, :1]
    hw = jnp.dot(f_ref[...], wg_ref[...], preferred_element_type=jnp.float32)
    t = hw * dinv
    out_ref[0, :, :] = t[:, :F2]
    out_ref[1, :, :] = t[:, F2:]


def _t2_body(s_ref, di_ref, bg_ref, wg_ref, out_ref):
    dinv = di_ref[...][:, :1]
    sfull = jnp.concatenate([s_ref[0], s_ref[1]], axis=1)
    h = jnp.maximum(sfull * dinv + bg_ref[...], 0.0)
    hw = jnp.dot(h, wg_ref[...], preferred_element_type=jnp.float32)
    t = hw * dinv
    out_ref[0, :, :] = t[:, :F2]
    out_ref[1, :, :] = t[:, F2:]


def _h2_body(s_ref, di_ref, bg_ref, out_ref):
    dinv = di_ref[...][:, :1]
    sfull = jnp.concatenate([s_ref[0], s_ref[1]], axis=1)
    h = jnp.maximum(sfull * dinv + bg_ref[...], 0.0)
    out_ref[0, :, :] = h[:, :F2]
    out_ref[1, :, :] = h[:, F2:]


def _cls_body(sums_ref, cntp_ref, wc1_ref, bc1_ref, wc2_ref, bc2_ref, out_ref):
    counts = jnp.sum(cntp_ref[...], axis=(0, 2)) * (1.0 / 128.0)
    pooled = jnp.concatenate([sums_ref[0], sums_ref[1]], axis=1)
    pooled = pooled / jnp.maximum(counts[:G], 1.0)[:, None]
    z = jnp.maximum(jnp.dot(pooled, wc1_ref[...], preferred_element_type=jnp.float32)
                    + bc1_ref[...], 0.0)
    out_ref[...] = jnp.dot(z, wc2_ref[...], preferred_element_type=jnp.float32) + bc2_ref[...]


def _full(shape):
    return pl.BlockSpec(shape, lambda *b: (0,) * len(shape))


def kernel(x, struct_emb, edge_index, batch, W1, b1, W2, b2, Wa, ba,
           Wg1, bg1, Wg2, bg2, Wc1, bc1, Wc2, bc2):
    f32 = jnp.float32
    i32 = jnp.int32

    src = jnp.concatenate([edge_index[0], jnp.zeros((EP - E,), i32)])
    dst1d = jnp.concatenate([edge_index[1], jnp.full((EP - E,), N, i32)])
    srcoff = jnp.concatenate([src, src + NP])
    bc1d = jnp.concatenate([batch, jnp.full((CNTP - N,), G, i32)])
    bp1d = jnp.concatenate([batch, jnp.full((NP - N,), G, i32)])
    ones_h = jnp.ones((128, 128), f32)
    zeros_h = jnp.zeros((128, 128), f32)
    b1r = b1.reshape(1, FUS)
    b2r = b2.reshape(1, FUS)
    war = Wa.reshape(1, FUS)
    bar = ba.reshape(1, 1)
    bg1r = bg1.reshape(1, HID)
    bg2r = bg2.reshape(1, HID)
    bc1r = bc1.reshape(1, HID // 2)
    bc2r = bc2.reshape(1, NC)

    BN = 400
    fused = pl.pallas_call(
        _fusion_body,
        grid=(N // BN,),
        in_specs=[
            pl.BlockSpec((BN, BERT), lambda b: (b, 0)),
            pl.BlockSpec((BN, N2V), lambda b: (b, 0)),
            _full((BERT, FUS)), _full((1, FUS)),
            _full((N2V, FUS)), _full((1, FUS)),
            _full((1, FUS)), _full((1, 1)),
        ],
        out_specs=pl.BlockSpec((BN, FUS), lambda b: (b, 0)),
        out_shape=jax.ShapeDtypeStruct((N, FUS), f32),
    )(x, struct_emb, W1, b1r, W2, b2r, war, bar)

    degp, cntp = _deg_kernel(dst1d, bc1d, ones_h, zeros_h)

    BM = 320
    grid = (NP // BM,)
    dp_spec = pl.BlockSpec((2, BM, 128), lambda b: (0, b, 0))
    dinv8 = pl.pallas_call(
        _dinv_body,
        grid=grid,
        in_specs=[dp_spec],
        out_specs=pl.BlockSpec((BM, 8), lambda b: (b, 0)),
        out_shape=jax.ShapeDtypeStruct((NP, 8), f32),
    )(degp)

    di_spec = pl.BlockSpec((BM, 8), lambda b: (b, 0))
    t3_spec = dict(
        out_specs=pl.BlockSpec((2, BM, F2), lambda b: (0, b, 0)),
        out_shape=jax.ShapeDtypeStruct((2, NP, F2), f32),
    )

    t3 = pl.pallas_call(
        _t1_body,
        grid=grid,
        in_specs=[pl.BlockSpec((BM, FUS), lambda b: (b, 0)), di_spec,
                  _full((FUS, HID))],
        **t3_spec,
    )(fused, dinv8, Wg1)

    s1 = _scatter_kernel(t3.reshape(2 * NP, F2), srcoff, dst1d)

    s_spec = pl.BlockSpec((2, BM, F2), lambda b: (0, b, 0))
    t3b = pl.pallas_call(
        _t2_body,
        grid=grid,
        in_specs=[s_spec, di_spec, _full((1, HID)), _full((HID, HID))],
        **t3_spec,
    )(s1, dinv8, bg1r, Wg2)

    s2 = _scatter_kernel(t3b.reshape(2 * NP, F2), srcoff, dst1d)

    h3 = pl.pallas_call(
        _h2_body,
        grid=grid,
        in_specs=[s_spec, di_spec, _full((1, HID))],
        **t3_spec,
    )(s2, dinv8, bg2r)

    sums = _pool_kernel(h3.reshape(2 * NP, F2), bp1d, zeros_h)

    out = pl.pallas_call(
        _cls_body,
        in_specs=[_full((2, G, F2)), _full((2, 128, 128)),
                  _full((HID, HID // 2)), _full((1, HID // 2)),
                  _full((HID // 2, NC)), _full((1, NC))],
        out_specs=_full((G, NC)),
        out_shape=jax.ShapeDtypeStruct((G, NC), f32),
    )(sums, cntp, Wc1, bc1r, Wc2, bc2r)
    return out


# speedup vs baseline: 1.0523x; 1.0523x over previous
import functools

import jax
import jax.numpy as jnp
from jax import lax
from jax.experimental import pallas as pl
from jax.experimental.pallas import tpu as pltpu
from jax.experimental.pallas import tpu_sc as plsc

N = 10000
E = 160000
G = 64
BERT = 768
N2V = 128
FUS = 256
HID = 256
NC = 10

NP = 10240
EP = 163840
CNTP = 12288
F2 = 128
NT = NP // 16
CHUNK = 128
CH = EP // 16 // CHUNK

_mesh = plsc.VectorSubcoreMesh(core_axis_name="c", subcore_axis_name="s")


@functools.partial(
    pl.kernel,
    out_type=(
        jax.ShapeDtypeStruct((2, NP, 128), jnp.float32),
        jax.ShapeDtypeStruct((2, 128, 128), jnp.float32),
    ),
    mesh=_mesh,
    scratch_types=[
        pltpu.VMEM((128,), jnp.int32),
        pltpu.VMEM((128, 128), jnp.float32),
        pltpu.VMEM((128, 128), jnp.float32),
        pltpu.VMEM_SHARED((NP, 128), jnp.float32),
        pltpu.VMEM_SHARED((128, 128), jnp.float32),
    ],
)
def _deg_kernel(dst1d, bc1d, ones_h, zeros_h, degp, cntp,
                ibuf, onesv, zerosv, dacc, cacc):
    c = lax.axis_index("c")
    s = lax.axis_index("s")
    w = c * 16 + s
    pltpu.sync_copy(ones_h, onesv)
    pltpu.sync_copy(zeros_h, zerosv)
    for k in range(NT // 128):
        pltpu.sync_copy(zerosv, dacc.at[pl.ds(s * NT + k * 128, 128)])

    @pl.when(s == 0)
    def _():
        pltpu.sync_copy(zerosv, cacc)

    plsc.subcore_barrier()

    def dbody(j, carry):
        pltpu.sync_copy(dst1d.at[pl.ds(w * (EP // 32) + j * 128, 128)], ibuf)
        pltpu.sync_copy(onesv, dacc.at[ibuf], add=True)
        return carry

    lax.fori_loop(0, EP // 32 // 128, dbody, 0)

    def cbody(j, carry):
        pltpu.sync_copy(bc1d.at[pl.ds(w * (CNTP // 32) + j * 128, 128)], ibuf)
        pltpu.sync_copy(onesv, cacc.at[ibuf], add=True)
        return carry

    lax.fori_loop(0, CNTP // 32 // 128, cbody, 0)
    plsc.subcore_barrier()
    pltpu.sync_copy(dacc.at[pl.ds(s * NT, NT)], degp.at[c, pl.ds(s * NT, NT)])

    @pl.when(s == 0)
    def _():
        pltpu.sync_copy(cacc, cntp.at[c])


NBUF = 2

@functools.partial(
    pl.kernel,
    out_type=jax.ShapeDtypeStruct((2, NP, F2), jnp.float32),
    mesh=_mesh,
    scratch_types=[
        [pltpu.VMEM((CHUNK,), jnp.int32) for _ in range(NBUF)],
        [pltpu.VMEM((CHUNK,), jnp.int32) for _ in range(NBUF)],
        [pltpu.VMEM((CHUNK, F2), jnp.float32) for _ in range(NBUF)],
        [pltpu.SemaphoreType.DMA for _ in range(NBUF)],
        [pltpu.SemaphoreType.DMA for _ in range(NBUF)],
        pltpu.VMEM_SHARED((NP, F2), jnp.float32),
    ],
)
def _scatter_kernel(tflat, srcoff, dst1d, s_out, sibuf, dibuf, rows, gsem, ssem, acc):
    c = lax.axis_index("c")
    s = lax.axis_index("s")
    ebase = s * (CH * CHUNK)

    def load_idx(b, j):
        pltpu.sync_copy(srcoff.at[pl.ds(c * EP + ebase + j * CHUNK, CHUNK)], sibuf[b])
        pltpu.sync_copy(dst1d.at[pl.ds(ebase + j * CHUNK, CHUNK)], dibuf[b])

    def gwait(b):
        pltpu.make_async_copy(tflat.at[sibuf[b]], rows[b], gsem[b]).wait()

    def swait(b):
        pltpu.make_async_copy(rows[b], acc.at[dibuf[b]], ssem[b]).wait()

    for b in range(NBUF):
        load_idx(b, b)
        pltpu.async_copy(tflat.at[sibuf[b]], rows[b], gsem[b])
    pltpu.sync_copy(tflat.at[pl.ds(c * NP + s * NT, NT)], acc.at[pl.ds(s * NT, NT)])
    plsc.subcore_barrier()

    def body(jj, carry):
        j = jj * NBUF
        for b in range(NBUF):
            gwait(b)
            pltpu.async_copy(rows[b], acc.at[dibuf[b]], ssem[b], add=True)
        for b in range(NBUF):
            swait(b)
            load_idx(b, j + b + NBUF)
            pltpu.async_copy(tflat.at[sibuf[b]], rows[b], gsem[b])
        return carry

    lax.fori_loop(0, CH // NBUF - 1, body, 0)
    for b in range(NBUF):
        gwait(b)
        pltpu.async_copy(rows[b], acc.at[dibuf[b]], ssem[b], add=True)
    for b in range(NBUF):
        swait(b)
    plsc.subcore_barrier()
    pltpu.sync_copy(acc.at[pl.ds(s * NT, NT)], s_out.at[c, pl.ds(s * NT, NT)])


@functools.partial(
    pl.kernel,
    out_type=jax.ShapeDtypeStruct((2, G, F2), jnp.float32),
    mesh=_mesh,
    scratch_types=[
        pltpu.VMEM((128,), jnp.int32),
        pltpu.VMEM((128, F2), jnp.float32),
        pltpu.VMEM_SHARED((128, F2), jnp.float32),
    ],
)
def _pool_kernel(hflat, bp1d, zeros_h, sums, ibuf, rows, pacc):
    c = lax.axis_index("c")
    s = lax.axis_index("s")

    @pl.when(s == 0)
    def _():
        pltpu.sync_copy(zeros_h, pacc)

    plsc.subcore_barrier()

    def body(j, carry):
        base = s * NT + j * 128
        pltpu.sync_copy(hflat.at[pl.ds(c * NP + base, 128)], rows)
        pltpu.sync_copy(bp1d.at[pl.ds(base, 128)], ibuf)
        pltpu.sync_copy(rows, pacc.at[ibuf], add=True)
        return carry

    lax.fori_loop(0, NT // 128, body, 0)
    plsc.subcore_barrier()

    @pl.when(s == 0)
    def _():
        pltpu.sync_copy(pacc.at[pl.ds(0, G)], sums.at[c])


def _fusion_body(x_ref, se_ref, w1_ref, b1_ref, w2_ref, b2_ref, wa_ref, ba_ref, out_ref):
    h1 = jnp.dot(x_ref[...], w1_ref[...], preferred_element_type=jnp.float32) + b1_ref[...]
    h2 = jnp.dot(se_ref[...], w2_ref[...], preferred_element_type=jnp.float32) + b2_ref[...]
    a1 = jnp.sum(h1 * wa_ref[...], axis=1, keepdims=True) + ba_ref[...]
    a2 = jnp.sum(h2 * wa_ref[...], axis=1, keepdims=True) + ba_ref[...]
    m = jnp.maximum(a1, a2)
    e1 = jnp.exp(a1 - m)
    e2 = jnp.exp(a2 - m)
    out_ref[...] = (e1 * h1 + e2 * h2) / (e1 + e2)


def _dinv_body(dp_ref, out_ref):
    deg = jnp.sum(dp_ref[...], axis=(0, 2)) * (1.0 / 128.0) + 1.0
    out_ref[...] = jnp.broadcast_to(lax.rsqrt(deg)[:, None], out_ref.shape)


def _t1_body(f_ref, di_ref, wg_ref, out_ref):
    dinv = di_ref[...][:, :1]
    hw = jnp.dot(f_ref[...], wg_ref[...], preferred_element_type=jnp.float32)
    t = hw * dinv
    out_ref[0, :, :] = t[:, :F2]
    out_ref[1, :, :] = t[:, F2:]


def _t2_body(s_ref, di_ref, bg_ref, wg_ref, out_ref):
    dinv = di_ref[...][:, :1]
    sfull = jnp.concatenate([s_ref[0], s_ref[1]], axis=1)
    h = jnp.maximum(sfull * dinv + bg_ref[...], 0.0)
    hw = jnp.dot(h, wg_ref[...], preferred_element_type=jnp.float32)
    t = hw * dinv
    out_ref[0, :, :] = t[:, :F2]
    out_ref[1, :, :] = t[:, F2:]


def _h2_body(s_ref, di_ref, bg_ref, out_ref):
    dinv = di_ref[...][:, :1]
    sfull = jnp.concatenate([s_ref[0], s_ref[1]], axis=1)
    h = jnp.maximum(sfull * dinv + bg_ref[...], 0.0)
    out_ref[0, :, :] = h[:, :F2]
    out_ref[1, :, :] = h[:, F2:]


def _cls_body(sums_ref, cntp_ref, wc1_ref, bc1_ref, wc2_ref, bc2_ref, out_ref):
    counts = jnp.sum(cntp_ref[...], axis=(0, 2)) * (1.0 / 128.0)
    pooled = jnp.concatenate([sums_ref[0], sums_ref[1]], axis=1)
    pooled = pooled / jnp.maximum(counts[:G], 1.0)[:, None]
    z = jnp.maximum(jnp.dot(pooled, wc1_ref[...], preferred_element_type=jnp.float32)
                    + bc1_ref[...], 0.0)
    out_ref[...] = jnp.dot(z, wc2_ref[...], preferred_element_type=jnp.float32) + bc2_ref[...]


def _full(shape):
    return pl.BlockSpec(shape, lambda *b: (0,) * len(shape))


def kernel(x, struct_emb, edge_index, batch, W1, b1, W2, b2, Wa, ba,
           Wg1, bg1, Wg2, bg2, Wc1, bc1, Wc2, bc2):
    f32 = jnp.float32
    i32 = jnp.int32

    src = jnp.concatenate([edge_index[0], jnp.zeros((EP - E,), i32)])
    dst1d = jnp.concatenate([edge_index[1], jnp.full((EP - E,), N, i32)])
    srcoff = jnp.concatenate([src, src + NP])
    bc1d = jnp.concatenate([batch, jnp.full((CNTP - N,), G, i32)])
    bp1d = jnp.concatenate([batch, jnp.full((NP - N,), G, i32)])
    ones_h = jnp.ones((128, 128), f32)
    zeros_h = jnp.zeros((128, 128), f32)
    b1r = b1.reshape(1, FUS)
    b2r = b2.reshape(1, FUS)
    war = Wa.reshape(1, FUS)
    bar = ba.reshape(1, 1)
    bg1r = bg1.reshape(1, HID)
    bg2r = bg2.reshape(1, HID)
    bc1r = bc1.reshape(1, HID // 2)
    bc2r = bc2.reshape(1, NC)

    BN = 400
    fused = pl.pallas_call(
        _fusion_body,
        grid=(N // BN,),
        in_specs=[
            pl.BlockSpec((BN, BERT), lambda b: (b, 0)),
            pl.BlockSpec((BN, N2V), lambda b: (b, 0)),
            _full((BERT, FUS)), _full((1, FUS)),
            _full((N2V, FUS)), _full((1, FUS)),
            _full((1, FUS)), _full((1, 1)),
        ],
        out_specs=pl.BlockSpec((BN, FUS), lambda b: (b, 0)),
        out_shape=jax.ShapeDtypeStruct((N, FUS), f32),
    )(x, struct_emb, W1, b1r, W2, b2r, war, bar)

    degp, cntp = _deg_kernel(dst1d, bc1d, ones_h, zeros_h)

    BM = 320
    grid = (NP // BM,)
    dp_spec = pl.BlockSpec((2, BM, 128), lambda b: (0, b, 0))
    dinv8 = pl.pallas_call(
        _dinv_body,
        grid=grid,
        in_specs=[dp_spec],
        out_specs=pl.BlockSpec((BM, 8), lambda b: (b, 0)),
        out_shape=jax.ShapeDtypeStruct((NP, 8), f32),
    )(degp)

    di_spec = pl.BlockSpec((BM, 8), lambda b: (b, 0))
    t3_spec = dict(
        out_specs=pl.BlockSpec((2, BM, F2), lambda b: (0, b, 0)),
        out_shape=jax.ShapeDtypeStruct((2, NP, F2), f32),
    )

    t3 = pl.pallas_call(
        _t1_body,
        grid=grid,
        in_specs=[pl.BlockSpec((BM, FUS), lambda b: (b, 0)), di_spec,
                  _full((FUS, HID))],
        **t3_spec,
    )(fused, dinv8, Wg1)

    s1 = _scatter_kernel(t3.reshape(2 * NP, F2), srcoff, dst1d)

    s_spec = pl.BlockSpec((2, BM, F2), lambda b: (0, b, 0))
    t3b = pl.pallas_call(
        _t2_body,
        grid=grid,
        in_specs=[s_spec, di_spec, _full((1, HID)), _full((HID, HID))],
        **t3_spec,
    )(s1, dinv8, bg1r, Wg2)

    s2 = _scatter_kernel(t3b.reshape(2 * NP, F2), srcoff, dst1d)

    h3 = pl.pallas_call(
        _h2_body,
        grid=grid,
        in_specs=[s_spec, di_spec, _full((1, HID))],
        **t3_spec,
    )(s2, dinv8, bg2r)

    sums = _pool_kernel(h3.reshape(2 * NP, F2), bp1d, zeros_h)

    out = pl.pallas_call(
        _cls_body,
        in_specs=[_full((2, G, F2)), _full((2, 128, 128)),
                  _full((HID, HID // 2)), _full((1, HID // 2)),
                  _full((HID // 2, NC)), _full((1, NC))],
        out_specs=_full((G, NC)),
        out_shape=jax.ShapeDtypeStruct((G, NC), f32),
    )(sums, cntp, Wc1, bc1r, Wc2, bc2r)
    return out


# speedup vs baseline: 1.0784x; 1.0249x over previous
import functools

import jax
import jax.numpy as jnp
from jax import lax
from jax.experimental import pallas as pl
from jax.experimental.pallas import tpu as pltpu
from jax.experimental.pallas import tpu_sc as plsc

N = 10000
E = 160000
G = 64
BERT = 768
N2V = 128
FUS = 256
HID = 256
NC = 10

NP = 10240
EP = 163840
CNTP = 12288
F2 = 128
NT = NP // 16
CHUNK = 128
CH = EP // 16 // CHUNK

_mesh = plsc.VectorSubcoreMesh(core_axis_name="c", subcore_axis_name="s")


@functools.partial(
    pl.kernel,
    out_type=(
        jax.ShapeDtypeStruct((2, NP, 128), jnp.float32),
        jax.ShapeDtypeStruct((2, 128, 128), jnp.float32),
    ),
    mesh=_mesh,
    scratch_types=[
        [pltpu.VMEM((128,), jnp.int32) for _ in range(2)],
        [pltpu.SemaphoreType.DMA for _ in range(2)],
        pltpu.VMEM((128, 128), jnp.float32),
        pltpu.VMEM((128, 128), jnp.float32),
        pltpu.VMEM_SHARED((NP, 128), jnp.float32),
        pltpu.VMEM_SHARED((128, 128), jnp.float32),
    ],
)
def _deg_kernel(dst1d, bc1d, ones_h, zeros_h, degp, cntp,
                ibuf, ssem, onesv, zerosv, dacc, cacc):
    c = lax.axis_index("c")
    s = lax.axis_index("s")
    w = c * 16 + s
    nd = EP // 32 // 128
    pltpu.sync_copy(ones_h, onesv)
    pltpu.sync_copy(zeros_h, zerosv)
    for k in range(NT // 128):
        pltpu.sync_copy(zerosv, dacc.at[pl.ds(s * NT + k * 128, 128)])

    @pl.when(s == 0)
    def _():
        pltpu.sync_copy(zerosv, cacc)

    plsc.subcore_barrier()

    def load(b, j):
        pltpu.sync_copy(dst1d.at[pl.ds(w * (EP // 32) + j * 128, 128)], ibuf[b])

    def start(b):
        pltpu.async_copy(onesv, dacc.at[ibuf[b]], ssem[b], add=True)

    def swait(b):
        pltpu.make_async_copy(onesv, dacc.at[ibuf[b]], ssem[b]).wait()

    load(0, 0)
    start(0)
    load(1, 1)
    start(1)

    def dbody(jj, carry):
        for b in range(2):
            swait(b)
            load(b, 2 * jj + 2 + b)
            start(b)
        return carry

    lax.fori_loop(0, nd // 2 - 1, dbody, 0)
    swait(0)
    swait(1)

    def cbody(j, carry):
        pltpu.sync_copy(bc1d.at[pl.ds(w * (CNTP // 32) + j * 128, 128)], ibuf[0])
        pltpu.sync_copy(onesv, cacc.at[ibuf[0]], add=True)
        return carry

    lax.fori_loop(0, CNTP // 32 // 128, cbody, 0)
    plsc.subcore_barrier()
    pltpu.sync_copy(dacc.at[pl.ds(s * NT, NT)], degp.at[c, pl.ds(s * NT, NT)])

    @pl.when(s == 0)
    def _():
        pltpu.sync_copy(cacc, cntp.at[c])


NBUF = 2

@functools.partial(
    pl.kernel,
    out_type=jax.ShapeDtypeStruct((2, NP, F2), jnp.float32),
    mesh=_mesh,
    scratch_types=[
        [pltpu.VMEM((CHUNK,), jnp.int32) for _ in range(NBUF)],
        [pltpu.VMEM((CHUNK,), jnp.int32) for _ in range(NBUF)],
        [pltpu.VMEM((CHUNK, F2), jnp.float32) for _ in range(NBUF)],
        [pltpu.SemaphoreType.DMA for _ in range(NBUF)],
        [pltpu.SemaphoreType.DMA for _ in range(NBUF)],
        pltpu.VMEM_SHARED((NP, F2), jnp.float32),
    ],
)
def _scatter_kernel(tflat, srcoff, dst1d, s_out, sibuf, dibuf, rows, gsem, ssem, acc):
    c = lax.axis_index("c")
    s = lax.axis_index("s")
    ebase = s * (CH * CHUNK)

    def load_idx(b, j):
        pltpu.sync_copy(srcoff.at[pl.ds(c * EP + ebase + j * CHUNK, CHUNK)], sibuf[b])
        pltpu.sync_copy(dst1d.at[pl.ds(ebase + j * CHUNK, CHUNK)], dibuf[b])

    def gwait(b):
        pltpu.make_async_copy(tflat.at[sibuf[b]], rows[b], gsem[b]).wait()

    def swait(b):
        pltpu.make_async_copy(rows[b], acc.at[dibuf[b]], ssem[b]).wait()

    for b in range(NBUF):
        load_idx(b, b)
        pltpu.async_copy(tflat.at[sibuf[b]], rows[b], gsem[b])
    pltpu.sync_copy(tflat.at[pl.ds(c * NP + s * NT, NT)], acc.at[pl.ds(s * NT, NT)])
    plsc.subcore_barrier()

    def body(jj, carry):
        j = jj * NBUF
        for b in range(NBUF):
            gwait(b)
            pltpu.async_copy(rows[b], acc.at[dibuf[b]], ssem[b], add=True)
        for b in range(NBUF):
            swait(b)
            load_idx(b, j + b + NBUF)
            pltpu.async_copy(tflat.at[sibuf[b]], rows[b], gsem[b])
        return carry

    lax.fori_loop(0, CH // NBUF - 1, body, 0)
    for b in range(NBUF):
        gwait(b)
        pltpu.async_copy(rows[b], acc.at[dibuf[b]], ssem[b], add=True)
    for b in range(NBUF):
        swait(b)
    plsc.subcore_barrier()
    pltpu.sync_copy(acc.at[pl.ds(s * NT, NT)], s_out.at[c, pl.ds(s * NT, NT)])


@functools.partial(
    pl.kernel,
    out_type=jax.ShapeDtypeStruct((2, G, F2), jnp.float32),
    mesh=_mesh,
    scratch_types=[
        pltpu.VMEM((128,), jnp.int32),
        pltpu.VMEM((128, F2), jnp.float32),
        pltpu.VMEM_SHARED((128, F2), jnp.float32),
    ],
)
def _pool_kernel(hflat, bp1d, zeros_h, sums, ibuf, rows, pacc):
    c = lax.axis_index("c")
    s = lax.axis_index("s")

    @pl.when(s == 0)
    def _():
        pltpu.sync_copy(zeros_h, pacc)

    plsc.subcore_barrier()

    def body(j, carry):
        base = s * NT + j * 128
        pltpu.sync_copy(hflat.at[pl.ds(c * NP + base, 128)], rows)
        pltpu.sync_copy(bp1d.at[pl.ds(base, 128)], ibuf)
        pltpu.sync_copy(rows, pacc.at[ibuf], add=True)
        return carry

    lax.fori_loop(0, NT // 128, body, 0)
    plsc.subcore_barrier()

    @pl.when(s == 0)
    def _():
        pltpu.sync_copy(pacc.at[pl.ds(0, G)], sums.at[c])


def _fusion_body(x_ref, se_ref, w1_ref, b1_ref, w2_ref, b2_ref, wa_ref, ba_ref, out_ref):
    h1 = jnp.dot(x_ref[...], w1_ref[...], preferred_element_type=jnp.float32) + b1_ref[...]
    h2 = jnp.dot(se_ref[...], w2_ref[...], preferred_element_type=jnp.float32) + b2_ref[...]
    a1 = jnp.sum(h1 * wa_ref[...], axis=1, keepdims=True) + ba_ref[...]
    a2 = jnp.sum(h2 * wa_ref[...], axis=1, keepdims=True) + ba_ref[...]
    m = jnp.maximum(a1, a2)
    e1 = jnp.exp(a1 - m)
    e2 = jnp.exp(a2 - m)
    out_ref[...] = (e1 * h1 + e2 * h2) / (e1 + e2)


def _dinv_body(dp_ref, out_ref):
    deg = jnp.sum(dp_ref[...], axis=(0, 2)) * (1.0 / 128.0) + 1.0
    out_ref[...] = jnp.broadcast_to(lax.rsqrt(deg)[:, None], out_ref.shape)


def _t1_body(f_ref, di_ref, wg_ref, out_ref):
    dinv = di_ref[...][:, :1]
    hw = jnp.dot(f_ref[...], wg_ref[...], preferred_element_type=jnp.float32)
    t = hw * dinv
    out_ref[0, :, :] = t[:, :F2]
    out_ref[1, :, :] = t[:, F2:]


def _t2_body(s_ref, di_ref, bg_ref, wg_ref, out_ref):
    dinv = di_ref[...][:, :1]
    sfull = jnp.concatenate([s_ref[0], s_ref[1]], axis=1)
    h = jnp.maximum(sfull * dinv + bg_ref[...], 0.0)
    hw = jnp.dot(h, wg_ref[...], preferred_element_type=jnp.float32)
    t = hw * dinv
    out_ref[0, :, :] = t[:, :F2]
    out_ref[1, :, :] = t[:, F2:]


def _h2_body(s_ref, di_ref, bg_ref, out_ref):
    dinv = di_ref[...][:, :1]
    sfull = jnp.concatenate([s_ref[0], s_ref[1]], axis=1)
    h = jnp.maximum(sfull * dinv + bg_ref[...], 0.0)
    out_ref[0, :, :] = h[:, :F2]
    out_ref[1, :, :] = h[:, F2:]


def _cls_body(sums_ref, cntp_ref, wc1_ref, bc1_ref, wc2_ref, bc2_ref, out_ref):
    counts = jnp.sum(cntp_ref[...], axis=(0, 2)) * (1.0 / 128.0)
    pooled = jnp.concatenate([sums_ref[0], sums_ref[1]], axis=1)
    pooled = pooled / jnp.maximum(counts[:G], 1.0)[:, None]
    z = jnp.maximum(jnp.dot(pooled, wc1_ref[...], preferred_element_type=jnp.float32)
                    + bc1_ref[...], 0.0)
    out_ref[...] = jnp.dot(z, wc2_ref[...], preferred_element_type=jnp.float32) + bc2_ref[...]


def _full(shape):
    return pl.BlockSpec(shape, lambda *b: (0,) * len(shape))


def kernel(x, struct_emb, edge_index, batch, W1, b1, W2, b2, Wa, ba,
           Wg1, bg1, Wg2, bg2, Wc1, bc1, Wc2, bc2):
    f32 = jnp.float32
    i32 = jnp.int32

    src = jnp.concatenate([edge_index[0], jnp.zeros((EP - E,), i32)])
    dst1d = jnp.concatenate([edge_index[1], jnp.full((EP - E,), N, i32)])
    srcoff = jnp.concatenate([src, src + NP])
    bc1d = jnp.concatenate([batch, jnp.full((CNTP - N,), G, i32)])
    bp1d = jnp.concatenate([batch, jnp.full((NP - N,), G, i32)])
    ones_h = jnp.ones((128, 128), f32)
    zeros_h = jnp.zeros((128, 128), f32)
    b1r = b1.reshape(1, FUS)
    b2r = b2.reshape(1, FUS)
    war = Wa.reshape(1, FUS)
    bar = ba.reshape(1, 1)
    bg1r = bg1.reshape(1, HID)
    bg2r = bg2.reshape(1, HID)
    bc1r = bc1.reshape(1, HID // 2)
    bc2r = bc2.reshape(1, NC)

    BN = 400
    fused = pl.pallas_call(
        _fusion_body,
        grid=(N // BN,),
        in_specs=[
            pl.BlockSpec((BN, BERT), lambda b: (b, 0)),
            pl.BlockSpec((BN, N2V), lambda b: (b, 0)),
            _full((BERT, FUS)), _full((1, FUS)),
            _full((N2V, FUS)), _full((1, FUS)),
            _full((1, FUS)), _full((1, 1)),
        ],
        out_specs=pl.BlockSpec((BN, FUS), lambda b: (b, 0)),
        out_shape=jax.ShapeDtypeStruct((N, FUS), f32),
    )(x, struct_emb, W1, b1r, W2, b2r, war, bar)

    degp, cntp = _deg_kernel(dst1d, bc1d, ones_h, zeros_h)

    BM = 320
    grid = (NP // BM,)
    dp_spec = pl.BlockSpec((2, BM, 128), lambda b: (0, b, 0))
    dinv8 = pl.pallas_call(
        _dinv_body,
        grid=grid,
        in_specs=[dp_spec],
        out_specs=pl.BlockSpec((BM, 8), lambda b: (b, 0)),
        out_shape=jax.ShapeDtypeStruct((NP, 8), f32),
    )(degp)

    di_spec = pl.BlockSpec((BM, 8), lambda b: (b, 0))
    t3_spec = dict(
        out_specs=pl.BlockSpec((2, BM, F2), lambda b: (0, b, 0)),
        out_shape=jax.ShapeDtypeStruct((2, NP, F2), f32),
    )

    t3 = pl.pallas_call(
        _t1_body,
        grid=grid,
        in_specs=[pl.BlockSpec((BM, FUS), lambda b: (b, 0)), di_spec,
                  _full((FUS, HID))],
        **t3_spec,
    )(fused, dinv8, Wg1)

    s1 = _scatter_kernel(t3.reshape(2 * NP, F2), srcoff, dst1d)

    s_spec = pl.BlockSpec((2, BM, F2), lambda b: (0, b, 0))
    t3b = pl.pallas_call(
        _t2_body,
        grid=grid,
        in_specs=[s_spec, di_spec, _full((1, HID)), _full((HID, HID))],
        **t3_spec,
    )(s1, dinv8, bg1r, Wg2)

    s2 = _scatter_kernel(t3b.reshape(2 * NP, F2), srcoff, dst1d)

    h3 = pl.pallas_call(
        _h2_body,
        grid=grid,
        in_specs=[s_spec, di_spec, _full((1, HID))],
        **t3_spec,
    )(s2, dinv8, bg2r)

    sums = _pool_kernel(h3.reshape(2 * NP, F2), bp1d, zeros_h)

    out = pl.pallas_call(
        _cls_body,
        in_specs=[_full((2, G, F2)), _full((2, 128, 128)),
                  _full((HID, HID // 2)), _full((1, HID // 2)),
                  _full((HID // 2, NC)), _full((1, NC))],
        out_specs=_full((G, NC)),
        out_shape=jax.ShapeDtypeStruct((G, NC), f32),
    )(sums, cntp, Wc1, bc1r, Wc2, bc2r)
    return out


# speedup vs baseline: 1.1232x; 1.0416x over previous
import functools

import jax
import jax.numpy as jnp
from jax import lax
from jax.experimental import pallas as pl
from jax.experimental.pallas import tpu as pltpu
from jax.experimental.pallas import tpu_sc as plsc

N = 10000
E = 160000
G = 64
BERT = 768
N2V = 128
FUS = 256
HID = 256
NC = 10

NP = 10240
EP = 163840
CNTP = 12288
F2 = 128
NT = NP // 16
CHUNK = 128
CH = EP // 16 // CHUNK

_mesh = plsc.VectorSubcoreMesh(core_axis_name="c", subcore_axis_name="s")


@functools.partial(
    pl.kernel,
    out_type=(
        jax.ShapeDtypeStruct((2, NP, 128), jnp.float32),
        jax.ShapeDtypeStruct((2, 128, 128), jnp.float32),
    ),
    mesh=_mesh,
    scratch_types=[
        [pltpu.VMEM((128,), jnp.int32) for _ in range(2)],
        [pltpu.SemaphoreType.DMA for _ in range(2)],
        pltpu.VMEM((128, 128), jnp.float32),
        pltpu.VMEM((128, 128), jnp.float32),
        pltpu.VMEM_SHARED((NP, 128), jnp.float32),
        pltpu.VMEM_SHARED((128, 128), jnp.float32),
    ],
)
def _deg_kernel(dst1d, bc1d, ones_h, zeros_h, degp, cntp,
                ibuf, ssem, onesv, zerosv, dacc, cacc):
    c = lax.axis_index("c")
    s = lax.axis_index("s")
    w = c * 16 + s
    nd = EP // 32 // 128
    pltpu.sync_copy(ones_h, onesv)
    pltpu.sync_copy(zeros_h, zerosv)
    for k in range(NT // 128):
        pltpu.sync_copy(zerosv, dacc.at[pl.ds(s * NT + k * 128, 128)])

    @pl.when(s == 0)
    def _():
        pltpu.sync_copy(zerosv, cacc)

    plsc.subcore_barrier()

    def load(b, j):
        pltpu.sync_copy(dst1d.at[pl.ds(w * (EP // 32) + j * 128, 128)], ibuf[b])

    def start(b):
        pltpu.async_copy(onesv, dacc.at[ibuf[b]], ssem[b], add=True)

    def swait(b):
        pltpu.make_async_copy(onesv, dacc.at[ibuf[b]], ssem[b]).wait()

    load(0, 0)
    start(0)
    load(1, 1)
    start(1)

    def dbody(jj, carry):
        for b in range(2):
            swait(b)
            load(b, 2 * jj + 2 + b)
            start(b)
        return carry

    lax.fori_loop(0, nd // 2 - 1, dbody, 0)
    swait(0)
    swait(1)

    def cbody(j, carry):
        pltpu.sync_copy(bc1d.at[pl.ds(w * (CNTP // 32) + j * 128, 128)], ibuf[0])
        pltpu.sync_copy(onesv, cacc.at[ibuf[0]], add=True)
        return carry

    lax.fori_loop(0, CNTP // 32 // 128, cbody, 0)
    plsc.subcore_barrier()
    pltpu.sync_copy(dacc.at[pl.ds(s * NT, NT)], degp.at[c, pl.ds(s * NT, NT)])

    @pl.when(s == 0)
    def _():
        pltpu.sync_copy(cacc, cntp.at[c])


NBUF = 2

@functools.partial(
    pl.kernel,
    out_type=jax.ShapeDtypeStruct((2, NP, F2), jnp.float32),
    mesh=_mesh,
    scratch_types=[
        [pltpu.VMEM((CHUNK,), jnp.int32) for _ in range(NBUF)],
        [pltpu.VMEM((CHUNK,), jnp.int32) for _ in range(NBUF)],
        [pltpu.VMEM((CHUNK, F2), jnp.float32) for _ in range(NBUF)],
        [pltpu.SemaphoreType.DMA for _ in range(NBUF)],
        pltpu.VMEM_SHARED((NP, F2), jnp.float32),
    ],
)
def _scatter_kernel(tflat, srcoff, dst1d, s_out, sibuf, dibuf, rows, gsem, acc):
    c = lax.axis_index("c")
    s = lax.axis_index("s")
    ebase = s * (CH * CHUNK)

    def load_idx(b, j):
        pltpu.sync_copy(srcoff.at[pl.ds(c * EP + ebase + j * CHUNK, CHUNK)], sibuf[b])
        pltpu.sync_copy(dst1d.at[pl.ds(ebase + j * CHUNK, CHUNK)], dibuf[b])

    def gwait(b):
        pltpu.make_async_copy(tflat.at[sibuf[b]], rows[b], gsem[b]).wait()

    for b in range(NBUF):
        load_idx(b, b)
        pltpu.async_copy(tflat.at[sibuf[b]], rows[b], gsem[b])
    pltpu.sync_copy(tflat.at[pl.ds(c * NP + s * NT, NT)], acc.at[pl.ds(s * NT, NT)])
    plsc.subcore_barrier()

    def body(jj, carry):
        for b in range(NBUF):
            gwait(b)
            pltpu.sync_copy(rows[b], acc.at[dibuf[b]], add=True)
            load_idx(b, jj * NBUF + b + NBUF)
            pltpu.async_copy(tflat.at[sibuf[b]], rows[b], gsem[b])
        return carry

    lax.fori_loop(0, CH // NBUF - 1, body, 0)
    for b in range(NBUF):
        gwait(b)
        pltpu.sync_copy(rows[b], acc.at[dibuf[b]], add=True)
    plsc.subcore_barrier()
    pltpu.sync_copy(acc.at[pl.ds(s * NT, NT)], s_out.at[c, pl.ds(s * NT, NT)])


@functools.partial(
    pl.kernel,
    out_type=jax.ShapeDtypeStruct((2, G, F2), jnp.float32),
    mesh=_mesh,
    scratch_types=[
        pltpu.VMEM((128,), jnp.int32),
        pltpu.VMEM((128, F2), jnp.float32),
        pltpu.VMEM_SHARED((128, F2), jnp.float32),
    ],
)
def _pool_kernel(hflat, bp1d, zeros_h, sums, ibuf, rows, pacc):
    c = lax.axis_index("c")
    s = lax.axis_index("s")

    @pl.when(s == 0)
    def _():
        pltpu.sync_copy(zeros_h, pacc)

    plsc.subcore_barrier()

    def body(j, carry):
        base = s * NT + j * 128
        pltpu.sync_copy(hflat.at[pl.ds(c * NP + base, 128)], rows)
        pltpu.sync_copy(bp1d.at[pl.ds(base, 128)], ibuf)
        pltpu.sync_copy(rows, pacc.at[ibuf], add=True)
        return carry

    lax.fori_loop(0, NT // 128, body, 0)
    plsc.subcore_barrier()

    @pl.when(s == 0)
    def _():
        pltpu.sync_copy(pacc.at[pl.ds(0, G)], sums.at[c])


def _fusion_body(x_ref, se_ref, w1_ref, b1_ref, w2_ref, b2_ref, wa_ref, ba_ref, out_ref):
    h1 = jnp.dot(x_ref[...], w1_ref[...], preferred_element_type=jnp.float32) + b1_ref[...]
    h2 = jnp.dot(se_ref[...], w2_ref[...], preferred_element_type=jnp.float32) + b2_ref[...]
    a1 = jnp.sum(h1 * wa_ref[...], axis=1, keepdims=True) + ba_ref[...]
    a2 = jnp.sum(h2 * wa_ref[...], axis=1, keepdims=True) + ba_ref[...]
    m = jnp.maximum(a1, a2)
    e1 = jnp.exp(a1 - m)
    e2 = jnp.exp(a2 - m)
    out_ref[...] = (e1 * h1 + e2 * h2) / (e1 + e2)


def _dinv_body(dp_ref, out_ref):
    deg = jnp.sum(dp_ref[...], axis=(0, 2)) * (1.0 / 128.0) + 1.0
    out_ref[...] = jnp.broadcast_to(lax.rsqrt(deg)[:, None], out_ref.shape)


def _t1_body(f_ref, di_ref, wg_ref, out_ref):
    dinv = di_ref[...][:, :1]
    hw = jnp.dot(f_ref[...], wg_ref[...], preferred_element_type=jnp.float32)
    t = hw * dinv
    out_ref[0, :, :] = t[:, :F2]
    out_ref[1, :, :] = t[:, F2:]


def _t2_body(s_ref, di_ref, bg_ref, wg_ref, out_ref):
    dinv = di_ref[...][:, :1]
    sfull = jnp.concatenate([s_ref[0], s_ref[1]], axis=1)
    h = jnp.maximum(sfull * dinv + bg_ref[...], 0.0)
    hw = jnp.dot(h, wg_ref[...], preferred_element_type=jnp.float32)
    t = hw * dinv
    out_ref[0, :, :] = t[:, :F2]
    out_ref[1, :, :] = t[:, F2:]


def _h2_body(s_ref, di_ref, bg_ref, out_ref):
    dinv = di_ref[...][:, :1]
    sfull = jnp.concatenate([s_ref[0], s_ref[1]], axis=1)
    h = jnp.maximum(sfull * dinv + bg_ref[...], 0.0)
    out_ref[0, :, :] = h[:, :F2]
    out_ref[1, :, :] = h[:, F2:]


def _cls_body(sums_ref, cntp_ref, wc1_ref, bc1_ref, wc2_ref, bc2_ref, out_ref):
    counts = jnp.sum(cntp_ref[...], axis=(0, 2)) * (1.0 / 128.0)
    pooled = jnp.concatenate([sums_ref[0], sums_ref[1]], axis=1)
    pooled = pooled / jnp.maximum(counts[:G], 1.0)[:, None]
    z = jnp.maximum(jnp.dot(pooled, wc1_ref[...], preferred_element_type=jnp.float32)
                    + bc1_ref[...], 0.0)
    out_ref[...] = jnp.dot(z, wc2_ref[...], preferred_element_type=jnp.float32) + bc2_ref[...]


def _full(shape):
    return pl.BlockSpec(shape, lambda *b: (0,) * len(shape))


def kernel(x, struct_emb, edge_index, batch, W1, b1, W2, b2, Wa, ba,
           Wg1, bg1, Wg2, bg2, Wc1, bc1, Wc2, bc2):
    f32 = jnp.float32
    i32 = jnp.int32

    src = jnp.concatenate([edge_index[0], jnp.zeros((EP - E,), i32)])
    dst1d = jnp.concatenate([edge_index[1], jnp.full((EP - E,), N, i32)])
    srcoff = jnp.concatenate([src, src + NP])
    bc1d = jnp.concatenate([batch, jnp.full((CNTP - N,), G, i32)])
    bp1d = jnp.concatenate([batch, jnp.full((NP - N,), G, i32)])
    ones_h = jnp.ones((128, 128), f32)
    zeros_h = jnp.zeros((128, 128), f32)
    b1r = b1.reshape(1, FUS)
    b2r = b2.reshape(1, FUS)
    war = Wa.reshape(1, FUS)
    bar = ba.reshape(1, 1)
    bg1r = bg1.reshape(1, HID)
    bg2r = bg2.reshape(1, HID)
    bc1r = bc1.reshape(1, HID // 2)
    bc2r = bc2.reshape(1, NC)

    BN = 400
    fused = pl.pallas_call(
        _fusion_body,
        grid=(N // BN,),
        in_specs=[
            pl.BlockSpec((BN, BERT), lambda b: (b, 0)),
            pl.BlockSpec((BN, N2V), lambda b: (b, 0)),
            _full((BERT, FUS)), _full((1, FUS)),
            _full((N2V, FUS)), _full((1, FUS)),
            _full((1, FUS)), _full((1, 1)),
        ],
        out_specs=pl.BlockSpec((BN, FUS), lambda b: (b, 0)),
        out_shape=jax.ShapeDtypeStruct((N, FUS), f32),
    )(x, struct_emb, W1, b1r, W2, b2r, war, bar)

    degp, cntp = _deg_kernel(dst1d, bc1d, ones_h, zeros_h)

    BM = 320
    grid = (NP // BM,)
    dp_spec = pl.BlockSpec((2, BM, 128), lambda b: (0, b, 0))
    dinv8 = pl.pallas_call(
        _dinv_body,
        grid=grid,
        in_specs=[dp_spec],
        out_specs=pl.BlockSpec((BM, 8), lambda b: (b, 0)),
        out_shape=jax.ShapeDtypeStruct((NP, 8), f32),
    )(degp)

    di_spec = pl.BlockSpec((BM, 8), lambda b: (b, 0))
    t3_spec = dict(
        out_specs=pl.BlockSpec((2, BM, F2), lambda b: (0, b, 0)),
        out_shape=jax.ShapeDtypeStruct((2, NP, F2), f32),
    )

    t3 = pl.pallas_call(
        _t1_body,
        grid=grid,
        in_specs=[pl.BlockSpec((BM, FUS), lambda b: (b, 0)), di_spec,
                  _full((FUS, HID))],
        **t3_spec,
    )(fused, dinv8, Wg1)

    s1 = _scatter_kernel(t3.reshape(2 * NP, F2), srcoff, dst1d)

    s_spec = pl.BlockSpec((2, BM, F2), lambda b: (0, b, 0))
    t3b = pl.pallas_call(
        _t2_body,
        grid=grid,
        in_specs=[s_spec, di_spec, _full((1, HID)), _full((HID, HID))],
        **t3_spec,
    )(s1, dinv8, bg1r, Wg2)

    s2 = _scatter_kernel(t3b.reshape(2 * NP, F2), srcoff, dst1d)

    h3 = pl.pallas_call(
        _h2_body,
        grid=grid,
        in_specs=[s_spec, di_spec, _full((1, HID))],
        **t3_spec,
    )(s2, dinv8, bg2r)

    sums = _pool_kernel(h3.reshape(2 * NP, F2), bp1d, zeros_h)

    out = pl.pallas_call(
        _cls_body,
        in_specs=[_full((2, G, F2)), _full((2, 128, 128)),
                  _full((HID, HID // 2)), _full((1, HID // 2)),
                  _full((HID // 2, NC)), _full((1, NC))],
        out_specs=_full((G, NC)),
        out_shape=jax.ShapeDtypeStruct((G, NC), f32),
    )(sums, cntp, Wc1, bc1r, Wc2, bc2r)
    return out
